# baseline (speedup 1.0000x reference)
import functools

import jax
import jax.numpy as jnp
from jax import lax
from jax.experimental import pallas as pl
from jax.experimental.pallas import tpu as pltpu
from jax.experimental.pallas import tpu_sc as plsc

BATCH = 16384
HIST = 20
DIM = 64
TOTAL = BATCH * HIST

_INFO = plsc.get_sparse_core_info()
NUM_CORES = _INFO.num_cores
NUM_SUBCORES = _INFO.num_subcores
NUM_WORKERS = NUM_CORES * NUM_SUBCORES

PER_WORKER = TOTAL // NUM_WORKERS
IDX_MINOR = 128
GATHERS_PER_GROUP = 4
GROUP_ROWS = IDX_MINOR * GATHERS_PER_GROUP
GROUPS = PER_WORKER // GROUP_ROWS
IDX_ROWS = PER_WORKER // IDX_MINOR

_mesh = plsc.VectorSubcoreMesh(core_axis_name="c", subcore_axis_name="s")


@functools.partial(
    pl.kernel,
    mesh=_mesh,
    out_type=jax.ShapeDtypeStruct((TOTAL, DIM), jnp.float32),
    scratch_types=[
        pltpu.VMEM((IDX_ROWS, IDX_MINOR), jnp.int32),
        pltpu.VMEM((GROUP_ROWS, DIM), jnp.float32),
        pltpu.SemaphoreType.DMA,
    ],
    compiler_params=pltpu.CompilerParams(use_tc_tiling_on_sc=False),
)
def _embed(table_hbm, idx_hbm, out_hbm, idx_v, rows_v, sem):
    wid = lax.axis_index("s") * NUM_CORES + lax.axis_index("c")
    base = wid * PER_WORKER
    pltpu.sync_copy(idx_hbm.at[wid], idx_v)

    def body(g, carry):
        handles = []
        for i in range(GATHERS_PER_GROUP):
            h = pltpu.async_copy(
                table_hbm.at[idx_v.at[g * GATHERS_PER_GROUP + i]],
                rows_v.at[pl.ds(i * IDX_MINOR, IDX_MINOR)],
                sem,
            )
            handles.append(h)
        for h in handles:
            h.wait()
        pltpu.sync_copy(
            rows_v, out_hbm.at[pl.ds(base + g * GROUP_ROWS, GROUP_ROWS)]
        )
        return carry

    lax.fori_loop(0, GROUPS, body, 0)


def kernel(input_, weight):
    idx = input_.reshape(NUM_WORKERS, IDX_ROWS, IDX_MINOR).astype(jnp.int32)
    out = _embed(weight, idx)
    return out.reshape(BATCH, HIST, DIM)


# speedup vs baseline: 1.0152x; 1.0152x over previous
import functools

import jax
import jax.numpy as jnp
from jax import lax
from jax.experimental import pallas as pl
from jax.experimental.pallas import tpu as pltpu
from jax.experimental.pallas import tpu_sc as plsc

BATCH = 16384
HIST = 20
DIM = 64
TOTAL = BATCH * HIST

_INFO = plsc.get_sparse_core_info()
NUM_CORES = _INFO.num_cores
NUM_SUBCORES = _INFO.num_subcores
NUM_WORKERS = NUM_CORES * NUM_SUBCORES

PER_WORKER = TOTAL // NUM_WORKERS
IDX_MINOR = 128
IDX_ROWS = PER_WORKER // IDX_MINOR
NBUF = 10
LAG = 6
NITER = IDX_ROWS // NBUF

_mesh = plsc.VectorSubcoreMesh(core_axis_name="c", subcore_axis_name="s")


@functools.partial(
    pl.kernel,
    mesh=_mesh,
    out_type=jax.ShapeDtypeStruct((TOTAL, DIM), jnp.float32),
    scratch_types=[
        pltpu.VMEM((IDX_ROWS, IDX_MINOR), jnp.int32),
        pltpu.VMEM((NBUF, IDX_MINOR, DIM), jnp.float32),
        pltpu.SemaphoreType.DMA((NBUF,)),
        pltpu.SemaphoreType.DMA((NBUF,)),
    ],
    compiler_params=pltpu.CompilerParams(use_tc_tiling_on_sc=False),
)
def _embed(table_hbm, idx_hbm, out_hbm, idx_v, bufs, sem_g, sem_o):
    wid = lax.axis_index("s") * NUM_CORES + lax.axis_index("c")
    base = wid * PER_WORKER
    pltpu.sync_copy(idx_hbm.at[wid], idx_v)

    def fire_gather(step, b):
        pltpu.async_copy(table_hbm.at[idx_v.at[step]], bufs.at[b], sem_g.at[b])

    def wait_gather(b):
        pltpu.make_async_copy(
            out_hbm.at[pl.ds(base, IDX_MINOR)], bufs.at[b], sem_g.at[b]
        ).wait()

    def fire_copyout(step, b):
        pltpu.async_copy(
            bufs.at[b],
            out_hbm.at[pl.ds(base + step * IDX_MINOR, IDX_MINOR)],
            sem_o.at[b],
        )

    def wait_copyout(b):
        pltpu.make_async_copy(
            bufs.at[b], out_hbm.at[pl.ds(base, IDX_MINOR)], sem_o.at[b]
        ).wait()

    for j in range(LAG):
        fire_gather(j, j)

    def body(i, carry):
        for k in range(NBUF):
            bg = (k + LAG) % NBUF
            if k < NBUF - LAG:
                @pl.when(i > 0)
                def _():
                    wait_copyout(bg)

                fire_gather(i * NBUF + k + LAG, bg)
            else:
                @pl.when(i < NITER - 1)
                def _():
                    wait_copyout(bg)
                    fire_gather(i * NBUF + k + LAG, bg)

            wait_gather(k)
            fire_copyout(i * NBUF + k, k)
        return carry

    lax.fori_loop(0, NITER, body, 0)

    for k in range(NBUF):
        wait_copyout(k)


def kernel(input_, weight):
    idx = input_.reshape(NUM_WORKERS, IDX_ROWS, IDX_MINOR).astype(jnp.int32)
    out = _embed(weight, idx)
    return out.reshape(BATCH, HIST, DIM)
